# baseline (speedup 1.0000x reference)
import functools
import math

import jax
import jax.numpy as jnp
from jax import lax
from jax.experimental import pallas as pl
from jax.experimental.pallas import tpu as pltpu
from jax.experimental.pallas import tpu_sc as plsc

NUM_FIELDS = 26
VOCAB = 100000
EMBED_DIM = 32
HIDDEN = 2048
BATCH = 4096
TOTAL_DIM = NUM_FIELDS * EMBED_DIM

_NC = 2
_NS = 16
_NW = _NC * _NS
_K_PER_W = TOTAL_DIM // _NW
_HALF0 = 50048
_HALF1 = VOCAB - _HALF0
_Q = (25088, 25088, 25088, 24736)
_QOFF = (0, 25088, 50176, 75264)


@functools.partial(
    pl.kernel,
    mesh=plsc.VectorSubcoreMesh(core_axis_name="c", subcore_axis_name="s"),
    compiler_params=pltpu.CompilerParams(needs_layout_passes=False),
    out_type=jax.ShapeDtypeStruct((TOTAL_DIM, BATCH), jnp.float32),
    scratch_types=[
        pltpu.VMEM((_Q[0],), jnp.float32),
        pltpu.VMEM((_Q[1],), jnp.float32),
        pltpu.VMEM((_Q[2],), jnp.float32),
        pltpu.VMEM((_Q[3],), jnp.float32),
        pltpu.VMEM((BATCH,), jnp.int32),
        pltpu.VMEM((BATCH,), jnp.float32),
        pltpu.VMEM((BATCH,), jnp.float32),
        pltpu.SemaphoreType.DMA,
        pltpu.SemaphoreType.DMA,
        pltpu.SemaphoreType.DMA,
        pltpu.SemaphoreType.DMA,
        pltpu.SemaphoreType.DMA,
    ],
)
def _sc_scan_gather(t3_hbm, idx_hbm, out_hbm, row_q0, row_q1, row_q2, row_q3,
                    idx_v, g_a, g_b, sem_q0, sem_q1, sem_q2, sem_q3, sem_o):
    rows = (row_q0, row_q1, row_q2, row_q3)
    sems = (sem_q0, sem_q1, sem_q2, sem_q3)
    wid = lax.axis_index("s") * _NC + lax.axis_index("c")
    k0 = wid * _K_PER_W

    def start_q(k, q):
        f = k // EMBED_DIM
        d = k % EMBED_DIM
        pltpu.async_copy(t3_hbm.at[f, d, pl.ds(_QOFF[q], _Q[q])], rows[q], sems[q])

    def wait_q(q):
        pltpu.make_async_copy(t3_hbm.at[0, 0, pl.ds(_QOFF[q], _Q[q])],
                              rows[q], sems[q]).wait()

    def gather_pass(buf, c0, ln, g_v, first):
        def body(i, c):
            ii = i * 16
            idx = idx_v[pl.ds(ii, 16)]
            local = idx if c0 == 0 else idx - c0
            local_u = plsc.bitcast(local, jnp.uint32)
            mask = local_u < jnp.uint32(ln)
            safe = plsc.bitcast(jnp.minimum(local_u, jnp.uint32(ln - 1)),
                                jnp.int32)
            vals = plsc.load_gather(buf, [safe])
            if first:
                g_v[pl.ds(ii, 16)] = vals
            else:
                g_v[pl.ds(ii, 16)] = jnp.where(mask, vals, g_v[pl.ds(ii, 16)])
            return c

        lax.fori_loop(0, BATCH // 16, body, 0, unroll=16)

    def per_pair(p, f_prev):
        def do_row(k, g_v, f_prev):
            f = k // EMBED_DIM

            @pl.when(f != f_prev)
            def _():
                pltpu.sync_copy(idx_hbm.at[f], idx_v)

            k_next = jnp.minimum(k + 1, TOTAL_DIM - 1)
            for q in range(4):
                wait_q(q)
                gather_pass(rows[q], _QOFF[q], _Q[q], g_v, q == 0)
                start_q(k_next, q)
            pltpu.async_copy(g_v, out_hbm.at[k], sem_o)
            return f

        @pl.when(p > 0)
        def _():
            pltpu.make_async_copy(g_a, out_hbm.at[0], sem_o).wait()
            pltpu.make_async_copy(g_b, out_hbm.at[0], sem_o).wait()

        k = k0 + 2 * p
        f_prev = do_row(k, g_a, f_prev)
        f_prev = do_row(k + 1, g_b, f_prev)
        return f_prev

    for q in range(4):
        start_q(k0, q)
    lax.fori_loop(0, _K_PER_W // 2, per_pair, jnp.int32(-1))
    for q in range(4):
        wait_q(q)
    pltpu.make_async_copy(g_a, out_hbm.at[0], sem_o).wait()
    pltpu.make_async_copy(g_b, out_hbm.at[0], sem_o).wait()


_BM = 256


def _tc_body(xt_ref, w_ref, b_ref, g_ref, be_ref, o_ref):
    xt = xt_ref[...]
    h = lax.dot_general(
        xt, w_ref[...],
        dimension_numbers=(((0,), (0,)), ((), ())),
        preferred_element_type=jnp.float32,
    )
    h = h + b_ref[...]
    mean = jnp.mean(h, axis=-1, keepdims=True)
    xc = h - mean
    var = jnp.mean(xc * xc, axis=-1, keepdims=True)
    hn = xc * lax.rsqrt(var + 1e-5)
    hn = hn * g_ref[...] + be_ref[...]
    o_ref[...] = hn * 0.5 * (1.0 + lax.erf(hn * (1.0 / math.sqrt(2.0))))


def _project(xT, W, b, gamma, beta):
    return pl.pallas_call(
        _tc_body,
        grid=(BATCH // _BM,),
        in_specs=[
            pl.BlockSpec((TOTAL_DIM, _BM), lambda i: (0, i)),
            pl.BlockSpec((TOTAL_DIM, HIDDEN), lambda i: (0, 0)),
            pl.BlockSpec((1, HIDDEN), lambda i: (0, 0)),
            pl.BlockSpec((1, HIDDEN), lambda i: (0, 0)),
            pl.BlockSpec((1, HIDDEN), lambda i: (0, 0)),
        ],
        out_specs=pl.BlockSpec((_BM, HIDDEN), lambda i: (i, 0)),
        out_shape=jax.ShapeDtypeStruct((BATCH, HIDDEN), jnp.float32),
    )(xT, W, b.reshape(1, HIDDEN),
      gamma.reshape(1, HIDDEN), beta.reshape(1, HIDDEN))


def kernel(indices, tables, W, b, gamma, beta):
    t3 = jnp.transpose(tables, (0, 2, 1))
    idxT = indices.astype(jnp.int32).T
    xT = _sc_scan_gather(t3, idxT)
    return _project(xT, W, b, gamma, beta)


# speedup vs baseline: 1.6546x; 1.6546x over previous
import functools
import math

import jax
import jax.numpy as jnp
from jax import lax
from jax.experimental import pallas as pl
from jax.experimental.pallas import tpu as pltpu
from jax.experimental.pallas import tpu_sc as plsc

NUM_FIELDS = 26
VOCAB = 100000
EMBED_DIM = 32
HIDDEN = 2048
BATCH = 4096
TOTAL_DIM = NUM_FIELDS * EMBED_DIM

_NC = 2
_NS = 16
_NW = _NC * _NS
_K_PER_W = TOTAL_DIM // _NW
_HALF0 = 50048
_HALF1 = VOCAB - _HALF0


@functools.partial(
    pl.kernel,
    mesh=plsc.VectorSubcoreMesh(core_axis_name="c", subcore_axis_name="s"),
    compiler_params=pltpu.CompilerParams(needs_layout_passes=False),
    out_type=jax.ShapeDtypeStruct((TOTAL_DIM, BATCH), jnp.float32),
    scratch_types=[
        pltpu.VMEM((_HALF0,), jnp.float32),
        pltpu.VMEM((_HALF1,), jnp.float32),
        pltpu.VMEM((BATCH,), jnp.int32),
        pltpu.VMEM((BATCH,), jnp.float32),
        pltpu.VMEM((BATCH,), jnp.float32),
        pltpu.SemaphoreType.DMA,
        pltpu.SemaphoreType.DMA,
        pltpu.SemaphoreType.DMA,
    ],
)
def _sc_scan_gather(t3_hbm, idx_hbm, out_hbm, row_a, row_b, idx_v, g_a, g_b,
                    sem_a, sem_b, sem_o):
    wid = lax.axis_index("s") * _NC + lax.axis_index("c")
    k0 = wid * _K_PER_W

    def start_half(k, half, buf, sem):
        f = k // EMBED_DIM
        d = k % EMBED_DIM
        off, ln = (0, _HALF0) if half == 0 else (_HALF0, _HALF1)
        pltpu.async_copy(t3_hbm.at[f, d, pl.ds(off, ln)], buf, sem)

    def wait_half(half, buf, sem):
        off, ln = (0, _HALF0) if half == 0 else (_HALF0, _HALF1)
        pltpu.make_async_copy(t3_hbm.at[0, 0, pl.ds(off, ln)], buf, sem).wait()

    def gather_pass(buf, c0, ln, g_v, first):
        def body(i, c):
            ii = i * 16
            idx = idx_v[pl.ds(ii, 16)]
            local = idx if c0 == 0 else idx - c0
            local_u = plsc.bitcast(local, jnp.uint32)
            mask = local_u < jnp.uint32(ln)
            safe = plsc.bitcast(jnp.minimum(local_u, jnp.uint32(ln - 1)),
                                jnp.int32)
            vals = plsc.load_gather(buf, [safe])
            if first:
                g_v[pl.ds(ii, 16)] = vals
            else:
                g_v[pl.ds(ii, 16)] = jnp.where(mask, vals, g_v[pl.ds(ii, 16)])
            return c

        lax.fori_loop(0, BATCH // 16, body, 0, unroll=16)

    def per_pair(p, f_prev):
        def do_row(k, g_v, f_prev):
            f = k // EMBED_DIM

            @pl.when(f != f_prev)
            def _():
                pltpu.sync_copy(idx_hbm.at[f], idx_v)

            wait_half(0, row_a, sem_a)
            gather_pass(row_a, 0, _HALF0, g_v, True)
            k_next = jnp.minimum(k + 1, TOTAL_DIM - 1)
            start_half(k_next, 0, row_a, sem_a)
            wait_half(1, row_b, sem_b)
            gather_pass(row_b, _HALF0, _HALF1, g_v, False)
            start_half(k_next, 1, row_b, sem_b)
            pltpu.async_copy(g_v, out_hbm.at[k], sem_o)
            return f

        @pl.when(p > 0)
        def _():
            pltpu.make_async_copy(g_a, out_hbm.at[0], sem_o).wait()
            pltpu.make_async_copy(g_b, out_hbm.at[0], sem_o).wait()

        k = k0 + 2 * p
        f_prev = do_row(k, g_a, f_prev)
        f_prev = do_row(k + 1, g_b, f_prev)
        return f_prev

    start_half(k0, 0, row_a, sem_a)
    start_half(k0, 1, row_b, sem_b)
    lax.fori_loop(0, _K_PER_W // 2, per_pair, jnp.int32(-1))
    wait_half(0, row_a, sem_a)
    wait_half(1, row_b, sem_b)
    pltpu.make_async_copy(g_a, out_hbm.at[0], sem_o).wait()
    pltpu.make_async_copy(g_b, out_hbm.at[0], sem_o).wait()


_BM = 256


def _tc_body(xt_ref, w_ref, b_ref, g_ref, be_ref, o_ref):
    xt = xt_ref[...]
    h = lax.dot_general(
        xt, w_ref[...],
        dimension_numbers=(((0,), (0,)), ((), ())),
        preferred_element_type=jnp.float32,
    )
    h = h + b_ref[...]
    mean = jnp.mean(h, axis=-1, keepdims=True)
    xc = h - mean
    var = jnp.mean(xc * xc, axis=-1, keepdims=True)
    hn = xc * lax.rsqrt(var + 1e-5)
    hn = hn * g_ref[...] + be_ref[...]
    o_ref[...] = hn * 0.5 * (1.0 + lax.erf(hn * (1.0 / math.sqrt(2.0))))


def _project(xT, W, b, gamma, beta):
    return pl.pallas_call(
        _tc_body,
        grid=(BATCH // _BM,),
        in_specs=[
            pl.BlockSpec((TOTAL_DIM, _BM), lambda i: (0, i)),
            pl.BlockSpec((TOTAL_DIM, HIDDEN), lambda i: (0, 0)),
            pl.BlockSpec((1, HIDDEN), lambda i: (0, 0)),
            pl.BlockSpec((1, HIDDEN), lambda i: (0, 0)),
            pl.BlockSpec((1, HIDDEN), lambda i: (0, 0)),
        ],
        out_specs=pl.BlockSpec((_BM, HIDDEN), lambda i: (i, 0)),
        out_shape=jax.ShapeDtypeStruct((BATCH, HIDDEN), jnp.float32),
    )(xT, W, b.reshape(1, HIDDEN),
      gamma.reshape(1, HIDDEN), beta.reshape(1, HIDDEN))


def kernel(indices, tables, W, b, gamma, beta):
    t3 = jnp.transpose(tables, (0, 2, 1))
    idxT = indices.astype(jnp.int32).T
    xT = _sc_scan_gather(t3, idxT)
    return _project(xT, W, b, gamma, beta)
